# Initial kernel scaffold; baseline (speedup 1.0000x reference)
import jax
import jax.numpy as jnp
from jax.experimental import pallas as pl


def kernel(input_, image, v_alpha, v_beta):
    raise NotImplementedError("write your pallas kernel here")


# speedup vs baseline: 2.1869x; 2.1869x over previous
import numpy as np
import jax
import jax.numpy as jnp
from jax.experimental import pallas as pl

_EPS = float(np.finfo(np.float64).eps)
_SIGMA_ALPHA = (8.0, 8.0, 8.0)
_SIGMA_BETA = 0.2
_BI = 256


def _pair_block_kernel(fi_ref, f2i_ref, ft_ref, f2j_ref, qf_ref, out_ref):
    fi = fi_ref[...]
    d2 = f2i_ref[...] + f2j_ref[...] - 2.0 * jax.lax.dot_general(
        fi, ft_ref[...], (((1,), (0,)), ((), ())),
        preferred_element_type=jnp.float32)
    wgt = jnp.exp(-0.5 * jnp.maximum(d2, 0.0))
    acc = jax.lax.dot_general(
        wgt, qf_ref[...], (((1,), (0,)), ((), ())),
        preferred_element_type=jnp.float32)
    c = out_ref.shape[1]
    out_ref[...] = acc[:, :c] * (1.0 / (acc[:, c:c + 1] + _EPS))


def kernel(input_, image, v_alpha, v_beta):
    C, d, h, w = input_.shape
    N = d * h * w

    z = jnp.arange(d, dtype=jnp.float32).reshape(-1, 1, 1)
    zz = v_alpha[0] * jnp.broadcast_to(z, (d, h, w)) / _SIGMA_ALPHA[0]
    x = jnp.arange(w, dtype=jnp.float32).reshape(1, 1, -1)
    xx = v_alpha[1] * jnp.broadcast_to(x, (d, h, w)) / _SIGMA_ALPHA[1]
    y = jnp.arange(h, dtype=jnp.float32).reshape(1, -1, 1)
    yy = v_alpha[2] * jnp.broadcast_to(y, (d, h, w)) / _SIGMA_ALPHA[2]
    xyz = jnp.stack([zz, yy, xx], axis=3)
    rgb = v_beta * jnp.transpose(image, (1, 2, 3, 0)) / float(_SIGMA_BETA)
    f = jnp.concatenate([xyz, rgb], axis=3).reshape(N, 6)

    f8 = jnp.pad(f, ((0, 0), (0, 2)))
    ft = f8.T
    f2 = jnp.sum(f * f, axis=1)
    f2col = f2.reshape(N, 1)
    f2row = f2.reshape(1, N)
    qf = jnp.concatenate(
        [input_.reshape(C, N), jnp.ones((1, N), jnp.float32)], axis=0).T

    out = pl.pallas_call(
        _pair_block_kernel,
        grid=(N // _BI,),
        in_specs=[
            pl.BlockSpec((_BI, 8), lambda i: (i, 0)),
            pl.BlockSpec((_BI, 1), lambda i: (i, 0)),
            pl.BlockSpec((8, N), lambda i: (0, 0)),
            pl.BlockSpec((1, N), lambda i: (0, 0)),
            pl.BlockSpec((N, C + 1), lambda i: (0, 0)),
        ],
        out_specs=pl.BlockSpec((_BI, C), lambda i: (i, 0)),
        out_shape=jax.ShapeDtypeStruct((N, C), jnp.float32),
    )(f8, f2col, ft, f2row, qf)
    return out.T.reshape(C, d, h, w)


# speedup vs baseline: 2.2647x; 1.0356x over previous
import numpy as np
import jax
import jax.numpy as jnp
from jax.experimental import pallas as pl

_EPS = float(np.finfo(np.float64).eps)
_SIGMA_ALPHA = (8.0, 8.0, 8.0)
_SIGMA_BETA = 0.2
_LOG2E = 1.4426950408889634
_BI = 256


def _pair_block_kernel(u_ref, wt_ref, qf_ref, out_ref):
    s = jax.lax.dot_general(
        u_ref[...], wt_ref[...], (((1,), (0,)), ((), ())),
        preferred_element_type=jnp.float32)
    wgt = jnp.exp2(jnp.minimum(s, 0.0))
    acc = jax.lax.dot_general(
        wgt, qf_ref[...], (((1,), (0,)), ((), ())),
        preferred_element_type=jnp.float32)
    c = out_ref.shape[1]
    out_ref[...] = acc[:, :c] * (1.0 / (acc[:, c:c + 1] + _EPS))


def kernel(input_, image, v_alpha, v_beta):
    C, d, h, w = input_.shape
    N = d * h * w

    z = jnp.arange(d, dtype=jnp.float32).reshape(-1, 1, 1)
    zz = v_alpha[0] * jnp.broadcast_to(z, (d, h, w)) / _SIGMA_ALPHA[0]
    x = jnp.arange(w, dtype=jnp.float32).reshape(1, 1, -1)
    xx = v_alpha[1] * jnp.broadcast_to(x, (d, h, w)) / _SIGMA_ALPHA[1]
    y = jnp.arange(h, dtype=jnp.float32).reshape(1, -1, 1)
    yy = v_alpha[2] * jnp.broadcast_to(y, (d, h, w)) / _SIGMA_ALPHA[2]
    xyz = jnp.stack([zz, yy, xx], axis=3)
    rgb = v_beta * jnp.transpose(image, (1, 2, 3, 0)) / float(_SIGMA_BETA)
    f = jnp.concatenate([xyz, rgb], axis=3).reshape(N, 6)

    f2 = jnp.sum(f * f, axis=1, keepdims=True)
    one = jnp.ones((N, 1), jnp.float32)
    u = _LOG2E * jnp.concatenate([f, f2, one], axis=1)
    wt = jnp.concatenate([f, -0.5 * one, -0.5 * f2], axis=1).T
    qf = jnp.concatenate(
        [input_.reshape(C, N), jnp.ones((1, N), jnp.float32)], axis=0).T

    out = pl.pallas_call(
        _pair_block_kernel,
        grid=(N // _BI,),
        in_specs=[
            pl.BlockSpec((_BI, 8), lambda i: (i, 0)),
            pl.BlockSpec((8, N), lambda i: (0, 0)),
            pl.BlockSpec((N, C + 1), lambda i: (0, 0)),
        ],
        out_specs=pl.BlockSpec((_BI, C), lambda i: (i, 0)),
        out_shape=jax.ShapeDtypeStruct((N, C), jnp.float32),
    )(u, wt, qf)
    return out.T.reshape(C, d, h, w)


# speedup vs baseline: 2.3238x; 1.0261x over previous
import numpy as np
import jax
import jax.numpy as jnp
from jax.experimental import pallas as pl

_EPS = float(np.finfo(np.float64).eps)
_SIGMA_ALPHA = (8.0, 8.0, 8.0)
_SIGMA_BETA = 0.2
_LOG2E = 1.4426950408889634
_BI = 256


def _pair_block_kernel(u_ref, wt_ref, qf_ref, out_ref):
    s = jax.lax.dot_general(
        u_ref[...], wt_ref[...], (((1,), (0,)), ((), ())),
        preferred_element_type=jnp.float32)
    wgt = jnp.exp2(jnp.minimum(s, 0.0))
    acc = jax.lax.dot_general(
        wgt.astype(jnp.bfloat16), qf_ref[...], (((1,), (0,)), ((), ())),
        preferred_element_type=jnp.float32)
    c = out_ref.shape[1]
    out_ref[...] = acc[:, :c] * (1.0 / (acc[:, c:c + 1] + _EPS))


def kernel(input_, image, v_alpha, v_beta):
    C, d, h, w = input_.shape
    N = d * h * w

    z = jnp.arange(d, dtype=jnp.float32).reshape(-1, 1, 1)
    zz = v_alpha[0] * jnp.broadcast_to(z, (d, h, w)) / _SIGMA_ALPHA[0]
    x = jnp.arange(w, dtype=jnp.float32).reshape(1, 1, -1)
    xx = v_alpha[1] * jnp.broadcast_to(x, (d, h, w)) / _SIGMA_ALPHA[1]
    y = jnp.arange(h, dtype=jnp.float32).reshape(1, -1, 1)
    yy = v_alpha[2] * jnp.broadcast_to(y, (d, h, w)) / _SIGMA_ALPHA[2]
    xyz = jnp.stack([zz, yy, xx], axis=3)
    rgb = v_beta * jnp.transpose(image, (1, 2, 3, 0)) / float(_SIGMA_BETA)
    f = jnp.concatenate([xyz, rgb], axis=3).reshape(N, 6)
    f = f - jnp.mean(f, axis=0, keepdims=True)

    f2 = jnp.sum(f * f, axis=1, keepdims=True)
    one = jnp.ones((N, 1), jnp.float32)
    u = (_LOG2E * jnp.concatenate([f, f2, one], axis=1)).astype(jnp.bfloat16)
    wt = jnp.concatenate(
        [f, -0.5 * one, -0.5 * f2], axis=1).T.astype(jnp.bfloat16)
    qf = jnp.concatenate(
        [input_.reshape(C, N), jnp.ones((1, N), jnp.float32)],
        axis=0).T.astype(jnp.bfloat16)

    out = pl.pallas_call(
        _pair_block_kernel,
        grid=(N // _BI,),
        in_specs=[
            pl.BlockSpec((_BI, 8), lambda i: (i, 0)),
            pl.BlockSpec((8, N), lambda i: (0, 0)),
            pl.BlockSpec((N, C + 1), lambda i: (0, 0)),
        ],
        out_specs=pl.BlockSpec((_BI, C), lambda i: (i, 0)),
        out_shape=jax.ShapeDtypeStruct((N, C), jnp.float32),
    )(u, wt, qf)
    return out.T.reshape(C, d, h, w)


# speedup vs baseline: 2.3994x; 1.0325x over previous
import numpy as np
import jax
import jax.numpy as jnp
from jax.experimental import pallas as pl

_EPS = float(np.finfo(np.float64).eps)
_SIGMA_ALPHA = (8.0, 8.0, 8.0)
_SIGMA_BETA = 0.2
_LOG2E = 1.4426950408889634
_BI = 512


def _pair_block_kernel(u_ref, wt_ref, qf_ref, out_ref):
    s = jax.lax.dot_general(
        u_ref[...], wt_ref[...], (((1,), (0,)), ((), ())),
        preferred_element_type=jnp.float32)
    wgt = jnp.exp2(jnp.minimum(s, 0.0))
    acc = jax.lax.dot_general(
        wgt.astype(jnp.bfloat16), qf_ref[...], (((1,), (0,)), ((), ())),
        preferred_element_type=jnp.float32)
    c = out_ref.shape[1]
    out_ref[...] = acc[:, :c] * (1.0 / (acc[:, c:c + 1] + _EPS))


def kernel(input_, image, v_alpha, v_beta):
    C, d, h, w = input_.shape
    N = d * h * w

    z = jnp.arange(d, dtype=jnp.float32).reshape(-1, 1, 1)
    zz = v_alpha[0] * jnp.broadcast_to(z, (d, h, w)) / _SIGMA_ALPHA[0]
    x = jnp.arange(w, dtype=jnp.float32).reshape(1, 1, -1)
    xx = v_alpha[1] * jnp.broadcast_to(x, (d, h, w)) / _SIGMA_ALPHA[1]
    y = jnp.arange(h, dtype=jnp.float32).reshape(1, -1, 1)
    yy = v_alpha[2] * jnp.broadcast_to(y, (d, h, w)) / _SIGMA_ALPHA[2]
    xyz = jnp.stack([zz, yy, xx], axis=3)
    rgb = v_beta * jnp.transpose(image, (1, 2, 3, 0)) / float(_SIGMA_BETA)
    f = jnp.concatenate([xyz, rgb], axis=3).reshape(N, 6)
    f = f - jnp.mean(f, axis=0, keepdims=True)

    f2 = jnp.sum(f * f, axis=1, keepdims=True)
    one = jnp.ones((N, 1), jnp.float32)
    u = (_LOG2E * jnp.concatenate([f, f2, one], axis=1)).astype(jnp.bfloat16)
    wt = jnp.concatenate(
        [f, -0.5 * one, -0.5 * f2], axis=1).T.astype(jnp.bfloat16)
    qf = jnp.concatenate(
        [input_.reshape(C, N), jnp.ones((1, N), jnp.float32)],
        axis=0).T.astype(jnp.bfloat16)

    out = pl.pallas_call(
        _pair_block_kernel,
        grid=(N // _BI,),
        in_specs=[
            pl.BlockSpec((_BI, 8), lambda i: (i, 0)),
            pl.BlockSpec((8, N), lambda i: (0, 0)),
            pl.BlockSpec((N, C + 1), lambda i: (0, 0)),
        ],
        out_specs=pl.BlockSpec((_BI, C), lambda i: (i, 0)),
        out_shape=jax.ShapeDtypeStruct((N, C), jnp.float32),
    )(u, wt, qf)
    return out.T.reshape(C, d, h, w)


# speedup vs baseline: 2.6125x; 1.0888x over previous
import numpy as np
import jax
import jax.numpy as jnp
from jax.experimental import pallas as pl

_EPS = float(np.finfo(np.float64).eps)
_SIGMA_ALPHA = (8.0, 8.0, 8.0)
_SIGMA_BETA = 0.2
_LOG2E = 1.4426950408889634
_BI = 512


def _pair_block_kernel(ut_ref, wt_ref, qft_ref, out_ref):
    u = jnp.transpose(ut_ref[...])
    s = jax.lax.dot_general(
        u, wt_ref[...], (((1,), (0,)), ((), ())),
        preferred_element_type=jnp.float32)
    wgt = jnp.exp2(jnp.minimum(s, 0.0))
    acc = jax.lax.dot_general(
        wgt.astype(jnp.bfloat16), qft_ref[...], (((1,), (1,)), ((), ())),
        preferred_element_type=jnp.float32)
    acc_t = jnp.transpose(acc)
    c = out_ref.shape[0]
    out_ref[...] = acc_t[:c, :] * (1.0 / (acc_t[c:c + 1, :] + _EPS))


def kernel(input_, image, v_alpha, v_beta):
    C, d, h, w = input_.shape
    N = d * h * w

    z = jnp.arange(d, dtype=jnp.float32).reshape(-1, 1, 1)
    zz = v_alpha[0] * jnp.broadcast_to(z, (d, h, w)) / _SIGMA_ALPHA[0]
    x = jnp.arange(w, dtype=jnp.float32).reshape(1, 1, -1)
    xx = v_alpha[1] * jnp.broadcast_to(x, (d, h, w)) / _SIGMA_ALPHA[1]
    y = jnp.arange(h, dtype=jnp.float32).reshape(1, -1, 1)
    yy = v_alpha[2] * jnp.broadcast_to(y, (d, h, w)) / _SIGMA_ALPHA[2]
    xyz = jnp.stack([zz, yy, xx], axis=0).reshape(3, N)
    rgb = (v_beta.reshape(1, 1) * image.reshape(3, N)) / float(_SIGMA_BETA)
    ft = jnp.concatenate([xyz, rgb], axis=0)
    ft = ft - jnp.mean(ft, axis=1, keepdims=True)

    f2t = jnp.sum(ft * ft, axis=0, keepdims=True)
    one = jnp.ones((1, N), jnp.float32)
    ut = (_LOG2E * jnp.concatenate([ft, f2t, one], axis=0)).astype(jnp.bfloat16)
    wt = jnp.concatenate(
        [ft, -0.5 * one, -0.5 * f2t], axis=0).astype(jnp.bfloat16)
    qft = jnp.concatenate(
        [input_.reshape(C, N), one], axis=0).astype(jnp.bfloat16)

    out = pl.pallas_call(
        _pair_block_kernel,
        grid=(N // _BI,),
        in_specs=[
            pl.BlockSpec((8, _BI), lambda i: (0, i)),
            pl.BlockSpec((8, N), lambda i: (0, 0)),
            pl.BlockSpec((C + 1, N), lambda i: (0, 0)),
        ],
        out_specs=pl.BlockSpec((C, _BI), lambda i: (0, i)),
        out_shape=jax.ShapeDtypeStruct((C, N), jnp.float32),
    )(ut, wt, qft)
    return out.reshape(C, d, h, w)


# speedup vs baseline: 2.7065x; 1.0360x over previous
import numpy as np
import jax
import jax.numpy as jnp
from jax.experimental import pallas as pl

_EPS = float(np.finfo(np.float64).eps)
_SIGMA_ALPHA = (8.0, 8.0, 8.0)
_SIGMA_BETA = 0.2
_LOG2E = 1.4426950408889634
_BI = 1024
_CJ = 2048


def _pair_block_kernel(ut_ref, wt_ref, qft_ref, out_ref):
    u = jnp.transpose(ut_ref[...])
    s = jax.lax.dot_general(
        u, wt_ref[...], (((1,), (0,)), ((), ())),
        preferred_element_type=jnp.float32)
    wgt = jnp.exp2(jnp.minimum(s, 0.0))
    acc = jax.lax.dot_general(
        wgt.astype(jnp.bfloat16), qft_ref[...], (((1,), (1,)), ((), ())),
        preferred_element_type=jnp.float32)
    acc_t = jnp.transpose(acc)
    c = out_ref.shape[0]
    out_ref[...] = acc_t[:c, :] * (1.0 / (acc_t[c:c + 1, :] + _EPS))


def kernel(input_, image, v_alpha, v_beta):
    C, d, h, w = input_.shape
    N = d * h * w

    z = jnp.arange(d, dtype=jnp.float32).reshape(-1, 1, 1)
    zz = v_alpha[0] * jnp.broadcast_to(z, (d, h, w)) / _SIGMA_ALPHA[0]
    x = jnp.arange(w, dtype=jnp.float32).reshape(1, 1, -1)
    xx = v_alpha[1] * jnp.broadcast_to(x, (d, h, w)) / _SIGMA_ALPHA[1]
    y = jnp.arange(h, dtype=jnp.float32).reshape(1, -1, 1)
    yy = v_alpha[2] * jnp.broadcast_to(y, (d, h, w)) / _SIGMA_ALPHA[2]
    xyz = jnp.stack([zz, yy, xx], axis=0).reshape(3, N)
    rgb = (v_beta.reshape(1, 1) * image.reshape(3, N)) / float(_SIGMA_BETA)
    ft = jnp.concatenate([xyz, rgb], axis=0)
    ft = ft - jnp.mean(ft, axis=1, keepdims=True)

    f2t = jnp.sum(ft * ft, axis=0, keepdims=True)
    one = jnp.ones((1, N), jnp.float32)
    ut = (_LOG2E * jnp.concatenate([ft, f2t, one], axis=0)).astype(jnp.bfloat16)
    wt = jnp.concatenate(
        [ft, -0.5 * one, -0.5 * f2t], axis=0).astype(jnp.bfloat16)
    qft = jnp.concatenate(
        [input_.reshape(C, N), one], axis=0).astype(jnp.bfloat16)

    out = pl.pallas_call(
        _pair_block_kernel,
        grid=(N // _BI,),
        in_specs=[
            pl.BlockSpec((8, _BI), lambda i: (0, i)),
            pl.BlockSpec((8, N), lambda i: (0, 0)),
            pl.BlockSpec((C + 1, N), lambda i: (0, 0)),
        ],
        out_specs=pl.BlockSpec((C, _BI), lambda i: (0, i)),
        out_shape=jax.ShapeDtypeStruct((C, N), jnp.float32),
    )(ut, wt, qft)
    return out.reshape(C, d, h, w)


# speedup vs baseline: 2.7111x; 1.0017x over previous
import numpy as np
import jax
import jax.numpy as jnp
from jax.experimental import pallas as pl

_EPS = float(np.finfo(np.float64).eps)
_SIGMA_ALPHA = (8.0, 8.0, 8.0)
_SIGMA_BETA = 0.2
_LOG2E = 1.4426950408889634
_BI = 1024
def _pair_block_kernel(ut_ref, wt_ref, qft_ref, out_ref):
    u = jnp.transpose(ut_ref[...])
    s = jax.lax.dot_general(
        u, wt_ref[...], (((1,), (0,)), ((), ())),
        preferred_element_type=jnp.float32)
    wgt = jnp.exp2(s.astype(jnp.bfloat16))
    acc = jax.lax.dot_general(
        wgt, qft_ref[...], (((1,), (1,)), ((), ())),
        preferred_element_type=jnp.float32)
    acc_t = jnp.transpose(acc)
    c = out_ref.shape[0]
    out_ref[...] = acc_t[:c, :] * (1.0 / (acc_t[c:c + 1, :] + _EPS))


def kernel(input_, image, v_alpha, v_beta):
    C, d, h, w = input_.shape
    N = d * h * w

    z = jnp.arange(d, dtype=jnp.float32).reshape(-1, 1, 1)
    zz = v_alpha[0] * jnp.broadcast_to(z, (d, h, w)) / _SIGMA_ALPHA[0]
    x = jnp.arange(w, dtype=jnp.float32).reshape(1, 1, -1)
    xx = v_alpha[1] * jnp.broadcast_to(x, (d, h, w)) / _SIGMA_ALPHA[1]
    y = jnp.arange(h, dtype=jnp.float32).reshape(1, -1, 1)
    yy = v_alpha[2] * jnp.broadcast_to(y, (d, h, w)) / _SIGMA_ALPHA[2]
    xyz = jnp.stack([zz, yy, xx], axis=0).reshape(3, N)
    rgb = (v_beta.reshape(1, 1) * image.reshape(3, N)) / float(_SIGMA_BETA)
    ft = jnp.concatenate([xyz, rgb], axis=0)
    ft = ft - jnp.mean(ft, axis=1, keepdims=True)

    f2t = jnp.sum(ft * ft, axis=0, keepdims=True)
    one = jnp.ones((1, N), jnp.float32)
    ut = (_LOG2E * jnp.concatenate([ft, f2t, one], axis=0)).astype(jnp.bfloat16)
    wt = jnp.concatenate(
        [ft, -0.5 * one, -0.5 * f2t], axis=0).astype(jnp.bfloat16)
    qft = jnp.concatenate(
        [input_.reshape(C, N), one], axis=0).astype(jnp.bfloat16)

    out = pl.pallas_call(
        _pair_block_kernel,
        grid=(N // _BI,),
        in_specs=[
            pl.BlockSpec((8, _BI), lambda i: (0, i)),
            pl.BlockSpec((8, N), lambda i: (0, 0)),
            pl.BlockSpec((C + 1, N), lambda i: (0, 0)),
        ],
        out_specs=pl.BlockSpec((C, _BI), lambda i: (0, i)),
        out_shape=jax.ShapeDtypeStruct((C, N), jnp.float32),
    )(ut, wt, qft)
    return out.reshape(C, d, h, w)
